# baseline (speedup 1.0000x reference)
import functools

import jax
import jax.numpy as jnp
from jax import lax
from jax.experimental import pallas as pl
from jax.experimental.pallas import tpu as pltpu
from jax.experimental.pallas import tpu_sc as plsc

_NEG_SLOPE = 0.2
_EPS = 1e-5

_NC = 2
_NS = 16
_NW = _NC * _NS


def _lrelu(v):
    return jnp.where(v >= 0, v, _NEG_SLOPE * v)


def _proj_body(x_ref, w_ref, pt_ref, qt_ref):
    xb = x_ref[0]
    y = lax.dot_general(
        xb, w_ref[...], (((0,), (0,)), ((), ())),
        preferred_element_type=jnp.float32,
    )
    pt_ref[...] = y[:, :64]
    qt_ref[...] = y[:, 64:]


def _project(x, wcat):
    B, F, N = x.shape
    bn = B * N
    nb = 2048
    nblk = N // nb
    return pl.pallas_call(
        _proj_body,
        grid=(B, nblk),
        in_specs=[
            pl.BlockSpec((1, F, nb), lambda b, i: (b, 0, i)),
            pl.BlockSpec((F, 128), lambda b, i: (0, 0)),
        ],
        out_specs=[
            pl.BlockSpec((nb, 64), lambda b, i: (b * nblk + i, 0)),
            pl.BlockSpec((nb, 64), lambda b, i: (b * nblk + i, 0)),
        ],
        out_shape=[
            jax.ShapeDtypeStruct((bn, 64), jnp.float32),
            jax.ShapeDtypeStruct((bn, 64), jnp.float32),
        ],
    )(x, wcat)


def _sc_gather_stats(pt, idx_w, qt):
    nw, kk, rpw = idx_w.shape
    bn = pt.shape[0]
    ch = 128
    nch = rpw // ch
    kpn = kk // 2
    nsteps = nch * kpn
    mesh = plsc.VectorSubcoreMesh(core_axis_name="c", subcore_axis_name="s")

    @functools.partial(
        pl.kernel,
        mesh=mesh,
        out_type=(
            jax.ShapeDtypeStruct((kpn, bn // 2, 128), jnp.int32),
            jax.ShapeDtypeStruct((nw, 8, 16), jnp.float32),
        ),
        scratch_types=[
            pltpu.VMEM((kk, rpw), jnp.int32),
            pltpu.VMEM((ch, 64), jnp.float32),
            pltpu.VMEM((ch, 64), jnp.float32),
            pltpu.VMEM((ch, 64), jnp.float32),
            pltpu.VMEM((ch, 64), jnp.float32),
            pltpu.VMEM((ch, 64), jnp.float32),
            pltpu.VMEM((ch // 2, 128), jnp.int32),
            pltpu.VMEM((ch // 2, 128), jnp.int32),
            pltpu.VMEM((8, 16), jnp.float32),
            pltpu.SemaphoreType.DMA,
            pltpu.SemaphoreType.DMA,
            pltpu.SemaphoreType.DMA,
            pltpu.SemaphoreType.DMA,
        ],
        compiler_params=pltpu.CompilerParams(
            use_tc_tiling_on_sc=False, needs_layout_passes=False
        ),
    )
    def k(pt_hbm, idxw_hbm, q_hbm, g_hbm, st_hbm, idx_v, q_v, ga0, gb0,
          ga1, gb1, pb0, pb1, st_v, gs0, gs1, ws0, ws1):
        wid = lax.axis_index("c") * _NS + lax.axis_index("s")
        base = wid * rpw
        base2 = (wid * rpw) // 2
        pltpu.sync_copy(idxw_hbm.at[wid], idx_v)

        def start_pair(s, ga, gb, sem):
            c = s // kpn
            k0 = 2 * lax.rem(s, kpn)
            roff = c * ch
            pltpu.async_copy(pt_hbm.at[idx_v.at[k0, pl.ds(roff, ch)]], ga, sem)
            pltpu.async_copy(
                pt_hbm.at[idx_v.at[k0 + 1, pl.ds(roff, ch)]], gb, sem
            )

        def drain_pair(ga, gb, sem):
            pltpu.make_async_copy(pt_hbm.at[pl.ds(0, ch)], ga, sem).wait()
            pltpu.make_async_copy(pt_hbm.at[pl.ds(0, ch)], gb, sem).wait()

        def start_write(s, pb, sem):
            c = s // kpn
            kpi = lax.rem(s, kpn)
            pltpu.async_copy(
                pb, g_hbm.at[kpi, pl.ds(base2 + c * (ch // 2), ch // 2)], sem
            )

        def drain_write(pb, sem):
            pltpu.make_async_copy(pb, g_hbm.at[0, pl.ds(0, ch // 2)], sem).wait()

        def maybe_load_q(s):
            @pl.when(lax.rem(s, kpn) == 0)
            def _():
                n0 = base // 2 + (s // kpn) * (ch // 2)
                pltpu.sync_copy(
                    q_hbm.at[0, pl.ds(n0, ch // 2)], q_v.at[pl.ds(0, ch // 2)]
                )
                pltpu.sync_copy(
                    q_hbm.at[1, pl.ds(n0, ch // 2)],
                    q_v.at[pl.ds(ch // 2, ch // 2)],
                )

        half = jnp.full((16,), 0x8000, jnp.int32)
        himask = jnp.full((16,), -65536, jnp.int32)

        def process(ga, gb, pb, carry):
            def row2(r2, cy):
                acc = list(cy)
                for parity in range(2):
                    r = 2 * r2 + parity
                    for j in range(4):
                        qv = q_v[parity * (ch // 2) + r2, pl.ds(16 * j, 16)]
                        y0 = ga[r, pl.ds(16 * j, 16)] + qv
                        y1 = gb[r, pl.ds(16 * j, 16)] + qv
                        acc[j] = acc[j] + (y0 + y1)
                        acc[4 + j] = acc[4 + j] + (y0 * y0 + y1 * y1)
                        u0 = plsc.bitcast(y0, jnp.int32)
                        u1 = plsc.bitcast(y1, jnp.int32)
                        w = lax.shift_right_logical(u0 + half, 16) | (
                            (u1 + half) & himask
                        )
                        pb[r2, pl.ds(parity * 64 + 16 * j, 16)] = w
                return tuple(acc)

            return lax.fori_loop(0, ch // 2, row2, carry)

        start_pair(0, ga0, gb0, gs0)

        def pair(t, carry):
            s0 = 2 * t

            @pl.when(t > 0)
            def _():
                drain_write(pb1, ws1)

            start_pair(s0 + 1, ga1, gb1, gs1)
            drain_pair(ga0, gb0, gs0)
            maybe_load_q(s0)
            carry = process(ga0, gb0, pb0, carry)
            start_write(s0, pb0, ws0)
            drain_write(pb0, ws0)

            @pl.when(t + 1 < nsteps // 2)
            def _():
                start_pair(s0 + 2, ga0, gb0, gs0)

            drain_pair(ga1, gb1, gs1)
            maybe_load_q(s0 + 1)
            carry = process(ga1, gb1, pb1, carry)
            start_write(s0 + 1, pb1, ws1)
            return carry

        zero = jnp.zeros((16,), jnp.float32)
        carry = lax.fori_loop(0, nsteps // 2, pair, (zero,) * 8)
        for j in range(8):
            st_v[j] = carry[j]
        pltpu.sync_copy(st_v, st_hbm.at[wid])
        drain_write(pb1, ws1)

    return k(pt, idx_w, qt)


def _main_body(g_ref, w2bd_ref, prm_ref, m_ref, s2_ref, q2_ref, *, kpn):
    sc1 = prm_ref[0:1, :]
    tc1 = prm_ref[1:2, :]
    w2bd = w2bd_ref[...]
    rb = m_ref.shape[0]
    m = jnp.full((rb, 128), -jnp.inf, jnp.float32)
    ssum = jnp.zeros((rb, 128), jnp.float32)
    ssq = jnp.zeros((rb, 128), jnp.float32)
    himask = jnp.int32(-65536)
    for kp in range(kpn):
        w = g_ref[kp]
        ylo = lax.bitcast_convert_type(lax.shift_left(w, 16), jnp.float32)
        yhi = lax.bitcast_convert_type(w & himask, jnp.float32)
        for y in (ylo, yhi):
            a = _lrelu(y * sc1 + tc1)
            y2 = jnp.dot(a, w2bd, preferred_element_type=jnp.float32)
            m = jnp.maximum(m, y2)
            ssum = ssum + y2
            ssq = ssq + y2 * y2
    m_ref[...] = m
    s2_ref[...] = jnp.broadcast_to(
        jnp.sum(ssum, axis=0)[None, None, :], (1, 8, 128)
    )
    q2_ref[...] = jnp.broadcast_to(
        jnp.sum(ssq, axis=0)[None, None, :], (1, 8, 128)
    )


def _main(g32, w2bd, prm):
    kpn, bn2, _ = g32.shape
    rb = 256
    nb = bn2 // rb
    return pl.pallas_call(
        functools.partial(_main_body, kpn=kpn),
        grid=(nb,),
        in_specs=[
            pl.BlockSpec((kpn, rb, 128), lambda i: (0, i, 0)),
            pl.BlockSpec((128, 128), lambda i: (0, 0)),
            pl.BlockSpec((8, 128), lambda i: (0, 0)),
        ],
        out_specs=[
            pl.BlockSpec((rb, 128), lambda i: (i, 0)),
            pl.BlockSpec((1, 8, 128), lambda i: (i, 0, 0)),
            pl.BlockSpec((1, 8, 128), lambda i: (i, 0, 0)),
        ],
        out_shape=[
            jax.ShapeDtypeStruct((bn2, 128), jnp.float32),
            jax.ShapeDtypeStruct((nb, 8, 128), jnp.float32),
            jax.ShapeDtypeStruct((nb, 8, 128), jnp.float32),
        ],
    )(g32, w2bd, prm)


def _epi_body(m_ref, prm_ref, eye_ref, o_ref):
    o = _lrelu(m_ref[...] * prm_ref[0:1, :] + prm_ref[1:2, :])
    eye = eye_ref[...]
    cd = (((1,), (1,)), ((), ()))
    o_ref[0] = lax.dot_general(eye, o[:, :64], cd, preferred_element_type=jnp.float32)
    o_ref[1] = lax.dot_general(eye, o[:, 64:], cd, preferred_element_type=jnp.float32)


def _epilogue(m, prm2):
    n = m.shape[0]
    rb = 1024
    eye = jnp.eye(64, dtype=jnp.float32)
    return pl.pallas_call(
        _epi_body,
        grid=(n // rb,),
        in_specs=[
            pl.BlockSpec((rb, 128), lambda i: (i, 0)),
            pl.BlockSpec((8, 128), lambda i: (0, 0)),
            pl.BlockSpec((64, 64), lambda i: (0, 0)),
        ],
        out_specs=pl.BlockSpec((2, 64, rb), lambda i: (0, 0, i)),
        out_shape=jax.ShapeDtypeStruct((2, 64, n), jnp.float32),
    )(m, prm2, eye)


def _pack_prm(sc, tc):
    row0 = jnp.concatenate([sc, sc])[None, :]
    row1 = jnp.concatenate([tc, tc])[None, :]
    return jnp.concatenate([row0, row1, jnp.zeros((6, 128), jnp.float32)], axis=0)


@jax.jit
def _impl(x, idx, W1, g1, b1, W2, g2, b2):
    B, F, N = x.shape
    K = idx.shape[-1]
    bn = B * N
    cnt = jnp.float32(B * N * K)

    w1a = W1[:, :F]
    w1b = W1[:, F:]
    wcat = jnp.concatenate([w1a.T, (w1b - w1a).T], axis=1)

    idx32 = idx.astype(jnp.int32) + (jnp.arange(B, dtype=jnp.int32) * N)[:, None, None]
    idx_t = jnp.transpose(idx32, (2, 1, 0)).reshape(K, bn)
    rpw = bn // _NW
    idx_w = jnp.transpose(idx_t.reshape(K, _NW, rpw), (1, 0, 2))

    pt, qt = _project(x, wcat)
    g32, st = _sc_gather_stats(pt, idx_w, qt.reshape(B, N, 64))

    s1 = jnp.sum(st[:, 0:4, :].reshape(_NW, 64), axis=0)
    s2 = jnp.sum(st[:, 4:8, :].reshape(_NW, 64), axis=0)
    m1 = s1 / cnt
    v1 = s2 / cnt - m1 * m1
    sc1 = g1 / jnp.sqrt(v1 + _EPS)
    tc1 = b1 - m1 * sc1

    w2t = W2.T
    zero64 = jnp.zeros((64, 64), jnp.float32)
    w2bd = jnp.block([[w2t, zero64], [zero64, w2t]])
    m, s2_p, q2_p = _main(g32, w2bd, _pack_prm(sc1, tc1))
    s2sum = jnp.sum(s2_p[:, 0, :], axis=0)
    s2sq = jnp.sum(q2_p[:, 0, :], axis=0)
    m2 = (s2sum[:64] + s2sum[64:]) / cnt
    ey2sq = (s2sq[:64] + s2sq[64:]) / cnt
    v2 = ey2sq - m2 * m2
    sc2 = g2 / jnp.sqrt(v2 + _EPS)
    tc2 = b2 - m2 * sc2

    return _epilogue(m, _pack_prm(sc2, tc2))


def kernel(x, fixed_knn_graph, W1, g1, b1, W2, g2, b2):
    return _impl(x, fixed_knn_graph, W1, g1, b1, W2, g2, b2)
